# Initial kernel scaffold; baseline (speedup 1.0000x reference)
import jax
import jax.numpy as jnp
from jax.experimental import pallas as pl


def kernel(x, edge_index, W1, b1, W2, b2):
    raise NotImplementedError("write your pallas kernel here")


# speedup vs baseline: 18.6368x; 18.6368x over previous
import functools

import jax
import jax.numpy as jnp
from jax import lax
from jax.experimental import pallas as pl
from jax.experimental.pallas import tpu as pltpu
from jax.experimental.pallas import tpu_sc as plsc

N = 10000
E = 320000
IN_DIM = 128
HID = 128
NUM_CLS = 40

NC = 2
NS = 16
NW = NC * NS
CH = 128
NCHUNK = 79
EPT = NCHUNK * CH
EPAD = NW * EPT
NPAD = 10240
RPS = NPAD // NS
BN = 2048

_MESH = plsc.VectorSubcoreMesh(
    core_axis_name="c", subcore_axis_name="s", num_cores=NC, num_subcores=NS
)


def _deg_body(dstp, degp, dst_v, deg_v):
    c = lax.axis_index("c")
    s = lax.axis_index("s")
    w = c * NS + s
    pltpu.sync_copy(dstp.at[w], dst_v)

    zeros16 = jnp.zeros((16,), jnp.float32)
    ones16 = jnp.ones((16,), jnp.float32)

    def zbody(i, carry):
        deg_v[pl.ds(i * 16, 16)] = zeros16
        return carry

    lax.fori_loop(0, NPAD // 16, zbody, 0)

    def ebody(i, carry):
        idx = dst_v[pl.ds(i * 16, 16)]
        plsc.addupdate_scatter(deg_v, [idx], ones16)
        return carry

    lax.fori_loop(0, EPT // 16, ebody, 0)
    pltpu.sync_copy(deg_v, degp.at[w])


_deg_call = functools.partial(
    pl.kernel,
    _deg_body,
    out_type=jax.ShapeDtypeStruct((NW, NPAD), jnp.float32),
    mesh=_MESH,
    compiler_params=pltpu.CompilerParams(needs_layout_passes=False, use_tc_tiling_on_sc=False),
    scratch_types=[
        pltpu.VMEM((EPT,), jnp.int32),
        pltpu.VMEM((NPAD,), jnp.float32),
    ],
)


def _edge_body(ht, srcp, dstp, out, src_v, dst_v, gbuf, acc_sh, gsem):
    c = lax.axis_index("c")
    s = lax.axis_index("s")
    w = c * NS + s
    pltpu.sync_copy(srcp.at[w], src_v)
    pltpu.sync_copy(dstp.at[w], dst_v)
    pltpu.sync_copy(ht.at[pl.ds(s * RPS, RPS)], acc_sh.at[pl.ds(s * RPS, RPS)])
    plsc.subcore_barrier()

    def ebody(j, carry):
        pltpu.async_copy(ht.at[src_v.at[j]], gbuf, gsem).wait()
        pltpu.sync_copy(gbuf, acc_sh.at[dst_v.at[j]], add=True)
        return carry

    lax.fori_loop(0, NCHUNK, ebody, 0)
    plsc.subcore_barrier()
    pltpu.sync_copy(
        acc_sh.at[pl.ds(s * RPS, RPS)], out.at[c, pl.ds(s * RPS, RPS)]
    )


def _make_edge_call(d):
    return functools.partial(
        pl.kernel,
        _edge_body,
        out_type=jax.ShapeDtypeStruct((NC, NPAD, d), jnp.float32),
        mesh=_MESH,
        compiler_params=pltpu.CompilerParams(needs_layout_passes=False, use_tc_tiling_on_sc=False),
        scratch_types=[
            pltpu.VMEM((NCHUNK, CH), jnp.int32),
            pltpu.VMEM((NCHUNK, CH), jnp.int32),
            pltpu.VMEM((CH, d), jnp.float32),
            pltpu.VMEM_SHARED((NPAD, d), jnp.float32),
            pltpu.SemaphoreType.DMA,
        ],
    )


_edge_call_1 = _make_edge_call(HID)
_edge_call_2 = _make_edge_call(NUM_CLS)


def _dinv_of(degp_block):
    return lax.rsqrt(jnp.sum(degp_block, axis=0) + 1.0)


def _mm1_body(x_ref, w_ref, degp_ref, o_ref):
    dinv = _dinv_of(degp_ref[...])
    o_ref[...] = (
        jnp.dot(x_ref[...], w_ref[...], preferred_element_type=jnp.float32)
        * dinv[:, None]
    )


def _mm2_body(acc_ref, ht_ref, degp_ref, b_ref, w_ref, o_ref):
    dinv = _dinv_of(degp_ref[...])
    a = acc_ref[0] + acc_ref[1] - ht_ref[...]
    h1 = jnp.maximum(a * dinv[:, None] + b_ref[...], 0.0)
    o_ref[...] = (
        jnp.dot(h1, w_ref[...], preferred_element_type=jnp.float32)
        * dinv[:, None]
    )


def _fin_body(acc_ref, ht_ref, degp_ref, b_ref, o_ref):
    dinv = _dinv_of(degp_ref[...])
    a = acc_ref[0] + acc_ref[1] - ht_ref[...]
    o_ref[...] = a * dinv[:, None] + b_ref[...]


def _row_spec(bn, d):
    return pl.BlockSpec((bn, d), lambda i: (i, 0))


def _degp_spec(bn):
    return pl.BlockSpec((NW, bn), lambda i: (0, i))


def _acc_spec(bn, d):
    return pl.BlockSpec((NC, bn, d), lambda i: (0, i, 0))


def _full_spec(*shape):
    return pl.BlockSpec(shape, lambda i: tuple(0 for _ in shape))


def kernel(x, edge_index, W1, b1, W2, b2):
    src = edge_index[0]
    dst = edge_index[1]
    pad_e = EPAD - E
    srcp = jnp.concatenate([src, jnp.zeros((pad_e,), src.dtype)])
    dstf = jnp.concatenate([dst, jnp.full((pad_e,), N, dst.dtype)])
    srcp3 = srcp.reshape(NW, NCHUNK, CH)
    dstp1 = dstf.reshape(NW, EPT)
    dstp3 = dstf.reshape(NW, NCHUNK, CH)
    xpad = jnp.concatenate(
        [x, jnp.zeros((NPAD - N, IN_DIM), x.dtype)], axis=0
    )
    b1r = b1.reshape(1, HID)
    b2r = b2.reshape(1, NUM_CLS)

    degp = _deg_call()(dstp1)

    ht1 = pl.pallas_call(
        _mm1_body,
        grid=(NPAD // BN,),
        in_specs=[
            _row_spec(BN, IN_DIM),
            _full_spec(IN_DIM, HID),
            _degp_spec(BN),
        ],
        out_specs=_row_spec(BN, HID),
        out_shape=jax.ShapeDtypeStruct((NPAD, HID), jnp.float32),
    )(xpad, W1, degp)

    acc1 = _edge_call_1()(ht1, srcp3, dstp3)

    ht2 = pl.pallas_call(
        _mm2_body,
        grid=(NPAD // BN,),
        in_specs=[
            _acc_spec(BN, HID),
            _row_spec(BN, HID),
            _degp_spec(BN),
            _full_spec(1, HID),
            _full_spec(HID, NUM_CLS),
        ],
        out_specs=_row_spec(BN, NUM_CLS),
        out_shape=jax.ShapeDtypeStruct((NPAD, NUM_CLS), jnp.float32),
    )(acc1, ht1, degp, b1r, W2)

    acc2 = _edge_call_2()(ht2, srcp3, dstp3)

    out = pl.pallas_call(
        _fin_body,
        grid=(NPAD // BN,),
        in_specs=[
            _acc_spec(BN, NUM_CLS),
            _row_spec(BN, NUM_CLS),
            _degp_spec(BN),
            _full_spec(1, NUM_CLS),
        ],
        out_specs=_row_spec(BN, NUM_CLS),
        out_shape=jax.ShapeDtypeStruct((NPAD, NUM_CLS), jnp.float32),
    )(acc2, ht2, degp, b2r)

    return out[:N]


# speedup vs baseline: 19.9932x; 1.0728x over previous
import functools

import jax
import jax.numpy as jnp
from jax import lax
from jax.experimental import pallas as pl
from jax.experimental.pallas import tpu as pltpu
from jax.experimental.pallas import tpu_sc as plsc

N = 10000
E = 320000
IN_DIM = 128
HID = 128
NUM_CLS = 40

NC = 2
NS = 16
NW = NC * NS
CH = 128
NCHUNK = 79
EPT = NCHUNK * CH
EPAD = NW * EPT
NPAD = 10240
RPS = NPAD // NS
BN = 2048

_MESH = plsc.VectorSubcoreMesh(
    core_axis_name="c", subcore_axis_name="s", num_cores=NC, num_subcores=NS
)


def _deg_body(dstp, degp, dst_v, deg_v):
    c = lax.axis_index("c")
    s = lax.axis_index("s")
    w = c * NS + s
    pltpu.sync_copy(dstp.at[w], dst_v)

    zeros16 = jnp.zeros((16,), jnp.float32)
    ones16 = jnp.ones((16,), jnp.float32)

    def zbody(i, carry):
        deg_v[pl.ds(i * 16, 16)] = zeros16
        return carry

    lax.fori_loop(0, NPAD // 16, zbody, 0)

    def ebody(i, carry):
        idx = dst_v[pl.ds(i * 16, 16)]
        plsc.addupdate_scatter(deg_v, [idx], ones16)
        return carry

    lax.fori_loop(0, EPT // 16, ebody, 0)
    pltpu.sync_copy(deg_v, degp.at[w])


_deg_call = functools.partial(
    pl.kernel,
    _deg_body,
    out_type=jax.ShapeDtypeStruct((NW, NPAD), jnp.float32),
    mesh=_MESH,
    compiler_params=pltpu.CompilerParams(needs_layout_passes=False, use_tc_tiling_on_sc=False),
    scratch_types=[
        pltpu.VMEM((EPT,), jnp.int32),
        pltpu.VMEM((NPAD,), jnp.float32),
    ],
)


NBUF = 2
NIB = 4


def _edge_body(ht, eidx, out, iring, gbuf, acc_sh, isem, gsem):
    c = lax.axis_index("c")
    s = lax.axis_index("s")
    w = c * NS + s
    pltpu.sync_copy(ht.at[pl.ds(s * RPS, RPS)], acc_sh.at[pl.ds(s * RPS, RPS)])

    for r in range(NIB - 1):
        pltpu.async_copy(eidx.at[w, r], iring.at[r], isem.at[r])
    plsc.subcore_barrier()
    pltpu.make_async_copy(eidx.at[w, 0], iring.at[0], isem.at[0]).wait()
    pltpu.async_copy(ht.at[iring.at[0, 0]], gbuf.at[0], gsem.at[0])

    def ebody(j, carry):
        b = lax.rem(j, NBUF)
        r = lax.rem(j, NIB)
        pltpu.make_async_copy(ht.at[iring.at[r, 0]], gbuf.at[b], gsem.at[b]).wait()
        ji = j + NIB - 1

        @pl.when(ji < NCHUNK)
        def _():
            ri = lax.rem(ji, NIB)
            pltpu.async_copy(eidx.at[w, ji], iring.at[ri], isem.at[ri])

        jg = j + 1

        @pl.when(jg < NCHUNK)
        def _():
            rg = lax.rem(jg, NIB)
            bg = lax.rem(jg, NBUF)
            pltpu.make_async_copy(
                eidx.at[w, jg], iring.at[rg], isem.at[rg]
            ).wait()
            pltpu.async_copy(ht.at[iring.at[rg, 0]], gbuf.at[bg], gsem.at[bg])

        pltpu.sync_copy(gbuf.at[b], acc_sh.at[iring.at[r, 1]], add=True)
        return carry

    lax.fori_loop(0, NCHUNK, ebody, 0)
    plsc.subcore_barrier()
    pltpu.sync_copy(
        acc_sh.at[pl.ds(s * RPS, RPS)], out.at[c, pl.ds(s * RPS, RPS)]
    )


def _make_edge_call(d):
    return functools.partial(
        pl.kernel,
        _edge_body,
        out_type=jax.ShapeDtypeStruct((NC, NPAD, d), jnp.float32),
        mesh=_MESH,
        compiler_params=pltpu.CompilerParams(needs_layout_passes=False, use_tc_tiling_on_sc=False),
        scratch_types=[
            pltpu.VMEM((NIB, 2, CH), jnp.int32),
            pltpu.VMEM((NBUF, CH, d), jnp.float32),
            pltpu.VMEM_SHARED((NPAD, d), jnp.float32),
            pltpu.SemaphoreType.DMA((NIB,)),
            pltpu.SemaphoreType.DMA((NBUF,)),
        ],
    )


_edge_call_1 = _make_edge_call(HID)
_edge_call_2 = _make_edge_call(NUM_CLS)


def _dinv_of(degp_block):
    return lax.rsqrt(jnp.sum(degp_block, axis=0) + 1.0)


def _mm1_body(x_ref, w_ref, degp_ref, o_ref):
    dinv = _dinv_of(degp_ref[...])
    o_ref[...] = (
        jnp.dot(x_ref[...], w_ref[...], preferred_element_type=jnp.float32)
        * dinv[:, None]
    )


def _mm2_body(acc_ref, ht_ref, degp_ref, b_ref, w_ref, o_ref):
    dinv = _dinv_of(degp_ref[...])
    a = acc_ref[0] + acc_ref[1] - ht_ref[...]
    h1 = jnp.maximum(a * dinv[:, None] + b_ref[...], 0.0)
    o_ref[...] = (
        jnp.dot(h1, w_ref[...], preferred_element_type=jnp.float32)
        * dinv[:, None]
    )


def _fin_body(acc_ref, ht_ref, degp_ref, b_ref, o_ref):
    dinv = _dinv_of(degp_ref[...])
    a = acc_ref[0] + acc_ref[1] - ht_ref[...]
    o_ref[...] = a * dinv[:, None] + b_ref[...]


def _row_spec(bn, d):
    return pl.BlockSpec((bn, d), lambda i: (i, 0))


def _degp_spec(bn):
    return pl.BlockSpec((NW, bn), lambda i: (0, i))


def _acc_spec(bn, d):
    return pl.BlockSpec((NC, bn, d), lambda i: (0, i, 0))


def _full_spec(*shape):
    return pl.BlockSpec(shape, lambda i: tuple(0 for _ in shape))


def kernel(x, edge_index, W1, b1, W2, b2):
    src = edge_index[0]
    dst = edge_index[1]
    pad_e = EPAD - E
    srcp = jnp.concatenate([src, jnp.zeros((pad_e,), src.dtype)])
    dstf = jnp.concatenate([dst, jnp.full((pad_e,), N, dst.dtype)])
    dstp1 = dstf.reshape(NW, EPT)
    eidx = jnp.stack(
        [srcp.reshape(NW, NCHUNK, CH), dstf.reshape(NW, NCHUNK, CH)], axis=2
    )
    xpad = jnp.concatenate(
        [x, jnp.zeros((NPAD - N, IN_DIM), x.dtype)], axis=0
    )
    b1r = b1.reshape(1, HID)
    b2r = b2.reshape(1, NUM_CLS)

    degp = _deg_call()(dstp1)

    ht1 = pl.pallas_call(
        _mm1_body,
        grid=(NPAD // BN,),
        in_specs=[
            _row_spec(BN, IN_DIM),
            _full_spec(IN_DIM, HID),
            _degp_spec(BN),
        ],
        out_specs=_row_spec(BN, HID),
        out_shape=jax.ShapeDtypeStruct((NPAD, HID), jnp.float32),
    )(xpad, W1, degp)

    acc1 = _edge_call_1()(ht1, eidx)

    ht2 = pl.pallas_call(
        _mm2_body,
        grid=(NPAD // BN,),
        in_specs=[
            _acc_spec(BN, HID),
            _row_spec(BN, HID),
            _degp_spec(BN),
            _full_spec(1, HID),
            _full_spec(HID, NUM_CLS),
        ],
        out_specs=_row_spec(BN, NUM_CLS),
        out_shape=jax.ShapeDtypeStruct((NPAD, NUM_CLS), jnp.float32),
    )(acc1, ht1, degp, b1r, W2)

    acc2 = _edge_call_2()(ht2, eidx)

    out = pl.pallas_call(
        _fin_body,
        grid=(NPAD // BN,),
        in_specs=[
            _acc_spec(BN, NUM_CLS),
            _row_spec(BN, NUM_CLS),
            _degp_spec(BN),
            _full_spec(1, NUM_CLS),
        ],
        out_specs=_row_spec(BN, NUM_CLS),
        out_shape=jax.ShapeDtypeStruct((NPAD, NUM_CLS), jnp.float32),
    )(acc2, ht2, degp, b2r)

    return out[:N]
